# baseline (speedup 1.0000x reference)
import functools

import jax
import jax.numpy as jnp
from jax import lax
from jax.experimental import pallas as pl
from jax.experimental.pallas import tpu as pltpu
from jax.experimental.pallas import tpu_sc as plsc

N = 10000
E = 320000
ND = 128
ED = 16
H = 128
L = 4
G = 64

NW = 32
EPW = E // NW
CK = 104
NCH = EPW // CK
CKT = EPW - NCH * CK
RSTG = 128
NRC = N // RSTG

_SQRT1_2 = 0.7071067811865476


def _gelu(x):
    return 0.5 * x * (1.0 + lax.erf(x * _SQRT1_2))


def _ln(x, g, b):
    mu = jnp.mean(x, axis=-1, keepdims=True)
    v = jnp.mean((x - mu) ** 2, axis=-1, keepdims=True)
    return (x - mu) * lax.rsqrt(v + 1e-5) * g + b


def _sc_gather(hw, src):
    mesh = plsc.VectorSubcoreMesh(core_axis_name="c", subcore_axis_name="s")

    @functools.partial(
        pl.kernel,
        out_type=jax.ShapeDtypeStruct((E, H), jnp.float32),
        mesh=mesh,
        scratch_types=[
            pltpu.VMEM((EPW,), jnp.int32),
            pltpu.VMEM((4, CK, H), jnp.float32),
            pltpu.VMEM((CKT, H), jnp.float32),
        ] + [pltpu.SemaphoreType.DMA] * 8,
    )
    def k(hw_hbm, src_hbm, out_hbm, idx_v, rows_v, tail_v, *sems):
        sem_g = sems[:4]
        sem_w = sems[4:]
        wid = lax.axis_index("s") * 2 + lax.axis_index("c")
        base = wid * EPW
        pltpu.sync_copy(src_hbm.at[pl.ds(base, EPW)], idx_v)

        def fire_gather(j, b):
            pltpu.async_copy(
                hw_hbm.at[idx_v.at[pl.ds(j * CK, CK)]], rows_v.at[b], sem_g[b])

        def drain_gather(j, b):
            pltpu.make_async_copy(
                hw_hbm.at[idx_v.at[pl.ds(j * CK, CK)]], rows_v.at[b],
                sem_g[b]).wait()

        def fire_write(j, b):
            pltpu.async_copy(
                rows_v.at[b], out_hbm.at[pl.ds(base + j * CK, CK)], sem_w[b])

        def drain_write(j, b):
            pltpu.make_async_copy(
                rows_v.at[b], out_hbm.at[pl.ds(base + j * CK, CK)],
                sem_w[b]).wait()

        fire_gather(0, 0)
        fire_gather(1, 1)

        def body(i, carry):
            for t in range(4):
                j = 4 * i + t
                b = t
                drain_gather(j, b)

                @pl.when(j >= 2)
                def _():
                    drain_write(j - 2, (b + 2) % 4)

                @pl.when(j + 2 < NCH)
                def _():
                    fire_gather(j + 2, (b + 2) % 4)

                fire_write(j, b)
            return carry

        lax.fori_loop(0, NCH // 4, body, 0)
        drain_write(NCH - 2, (NCH - 2) % 4)
        drain_write(NCH - 1, (NCH - 1) % 4)

        toff = NCH * CK
        pltpu.async_copy(
            hw_hbm.at[idx_v.at[pl.ds(toff, CKT)]], tail_v, sem_g[0]).wait()
        pltpu.sync_copy(tail_v, out_hbm.at[pl.ds(base + toff, CKT)])

    return k(hw, src)


FCK = 80
FNCH = EPW // FCK
IGRP = 25
NVEC = FCK * H // 16
TAB = 1024
TABP = 1032


def _sc_fused(hw, ewb, src, dst, zstage):
    mesh = plsc.VectorSubcoreMesh(core_axis_name="c", subcore_axis_name="s")

    @functools.partial(
        pl.kernel,
        out_type=jax.ShapeDtypeStruct((2, N, H), jnp.float32),
        mesh=mesh,
        scratch_types=[
            pltpu.VMEM((2 * IGRP * FCK,), jnp.int32),
            pltpu.VMEM((2 * IGRP * FCK,), jnp.int32),
            pltpu.VMEM((2, FCK, H), jnp.float32),
            pltpu.VMEM((2, FCK, H), jnp.float32),
            pltpu.VMEM_SHARED((N, H), jnp.float32),
        ] + [pltpu.SemaphoreType.DMA] * 6,
    )
    def k(hw_hbm, ew_hbm, src_hbm, dst_hbm, z_hbm, out_hbm,
          sidx_v, didx_v, hwb_v, ewb_v, acc, *sems):
        sem_g = sems[0:2]
        sem_e = sems[2:4]
        sem_a = sems[4:6]
        c = lax.axis_index("c")
        s = lax.axis_index("s")
        wid = s * 2 + c
        base = wid * EPW

        for t in range(5):
            rc = t * 16 + s

            @pl.when(rc < NRC)
            def _():
                pltpu.sync_copy(z_hbm, acc.at[pl.ds(rc * RSTG, RSTG)])

        @pl.when(s == 0)
        def _():
            pltpu.sync_copy(z_hbm.at[pl.ds(0, N - NRC * RSTG)],
                            acc.at[pl.ds(NRC * RSTG, N - NRC * RSTG)])

        plsc.subcore_barrier()

        GW = IGRP * FCK

        def load_groups(g):
            s0 = lax.rem(g, 2) * GW
            pltpu.sync_copy(src_hbm.at[pl.ds(base + g * GW, GW)],
                            sidx_v.at[pl.ds(s0, GW)])
            pltpu.sync_copy(dst_hbm.at[pl.ds(base + g * GW, GW)],
                            didx_v.at[pl.ds(s0, GW)])

        def _ioff(j):
            return lax.rem(j // IGRP, 2) * GW + lax.rem(j, IGRP) * FCK

        def fire_gather(j, b):
            pltpu.async_copy(
                hw_hbm.at[sidx_v.at[pl.ds(_ioff(j), FCK)]], hwb_v.at[b],
                sem_g[b])
            pltpu.async_copy(
                ew_hbm.at[pl.ds(base + j * FCK, FCK)], ewb_v.at[b], sem_e[b])

        def drain_gather(j, b):
            pltpu.make_async_copy(
                hw_hbm.at[sidx_v.at[pl.ds(_ioff(j), FCK)]], hwb_v.at[b],
                sem_g[b]).wait()
            pltpu.make_async_copy(
                ew_hbm.at[pl.ds(base + j * FCK, FCK)], ewb_v.at[b],
                sem_e[b]).wait()

        def fire_add(j, b):
            pltpu.async_copy(
                hwb_v.at[b],
                acc.at[didx_v.at[pl.ds(_ioff(j), FCK)]], sem_a[b], add=True)

        def drain_add(j, b):
            pltpu.make_async_copy(
                hwb_v.at[b],
                acc.at[didx_v.at[pl.ds(_ioff(j), FCK)]], sem_a[b]).wait()

        def compute(b):
            hwb_b = hwb_v.at[b]
            ewb_b = ewb_v.at[b]

            def vec_body(q, carry):
                row = q // 8
                col = (q % 8) * 16
                x = hwb_b[row, pl.ds(col, 16)] + ewb_b[row, pl.ds(col, 16)]
                u = x * (1.5957691216 + 0.0713548162726 * (x * x))
                u = jnp.minimum(u, 60.0)
                ex = jnp.exp(u)
                y = x * ex / (ex + 1.0)
                hwb_b[row, pl.ds(col, 16)] = y
                return carry

            lax.fori_loop(0, NVEC, vec_body, 0)

        def step(j, b):
            drain_gather(j, b)

            @pl.when(j >= 1)
            def _():
                drain_add(j - 1, 1 - b)

            @pl.when((lax.rem(j, IGRP) == IGRP - 1) & (j + 1 < FNCH))
            def _():
                load_groups((j + 1) // IGRP)

            @pl.when(j + 1 < FNCH)
            def _():
                fire_gather(j + 1, 1 - b)

            compute(b)
            fire_add(j, b)

        load_groups(0)
        fire_gather(0, 0)

        def body(i, carry):
            step(2 * i, 0)
            step(2 * i + 1, 1)
            return carry

        lax.fori_loop(0, FNCH // 2, body, 0)
        step(FNCH - 1, 0)
        drain_add(FNCH - 1, 0)
        plsc.subcore_barrier()

        for t in range(5):
            rc = t * 16 + s

            @pl.when(rc < NRC)
            def _():
                r0 = rc * RSTG
                pltpu.sync_copy(acc.at[pl.ds(r0, RSTG)],
                                out_hbm.at[c, pl.ds(r0, RSTG)])

        @pl.when(s == 1)
        def _():
            pltpu.sync_copy(acc.at[pl.ds(NRC * RSTG, N - NRC * RSTG)],
                            out_hbm.at[c, pl.ds(NRC * RSTG, N - NRC * RSTG)])

    return k(hw, ewb, src, dst, zstage)


def _tc_embed(x, emb_W, emb_b, emb_g, emb_bb, Wh0):
    NB = 1000
    grid = (N // NB,)

    def body(x_ref, w_ref, b_ref, g_ref, bb_ref, wh_ref, h_ref, hw_ref):
        h = jnp.dot(x_ref[...], w_ref[...], preferred_element_type=jnp.float32)
        h = _gelu(_ln(h + b_ref[...], g_ref[...], bb_ref[...]))
        h_ref[...] = h
        hw_ref[...] = jnp.dot(h, wh_ref[...], preferred_element_type=jnp.float32)

    full = lambda i: (0, 0)
    return pl.pallas_call(
        body,
        grid=grid,
        in_specs=[
            pl.BlockSpec((NB, ND), lambda i: (i, 0)),
            pl.BlockSpec((ND, H), full),
            pl.BlockSpec((1, H), full),
            pl.BlockSpec((1, H), full),
            pl.BlockSpec((1, H), full),
            pl.BlockSpec((H, H), full),
        ],
        out_specs=[
            pl.BlockSpec((NB, H), lambda i: (i, 0)),
            pl.BlockSpec((NB, H), lambda i: (i, 0)),
        ],
        out_shape=[
            jax.ShapeDtypeStruct((N, H), jnp.float32),
            jax.ShapeDtypeStruct((N, H), jnp.float32),
        ],
    )(x, emb_W, emb_b.reshape(1, H), emb_g.reshape(1, H), emb_bb.reshape(1, H), Wh0)


def _tc_ewb(ea, We, cb):
    EB = 2000
    grid = (E // EB,)

    def body(ea_ref, we_ref, cb_ref, m_ref):
        m_ref[...] = jnp.dot(ea_ref[...], we_ref[...],
                             preferred_element_type=jnp.float32) + cb_ref[...]

    full = lambda i: (0, 0)
    return pl.pallas_call(
        body,
        grid=grid,
        in_specs=[
            pl.BlockSpec((EB, ED), lambda i: (i, 0)),
            pl.BlockSpec((ED, H), full),
            pl.BlockSpec((1, H), full),
        ],
        out_specs=pl.BlockSpec((EB, H), lambda i: (i, 0)),
        out_shape=jax.ShapeDtypeStruct((E, H), jnp.float32),
    )(ea, We, cb.reshape(1, H))


def _tc_update(h, a2, batch2, ng, nb, Wh_next):
    NB = 1000
    grid = (N // NB,)

    def body(h_ref, a_ref, b_ref, g_ref, bb_ref, wh_ref, hn_ref, hw_ref,
             rep_ref, cnt_ref):
        pid = pl.program_id(0)
        a = a_ref[0] + a_ref[1]
        hn = _ln(h_ref[...] + a, g_ref[...], bb_ref[...])
        hn_ref[...] = hn
        hw_ref[...] = jnp.dot(hn, wh_ref[...], preferred_element_type=jnp.float32)
        seg = lax.broadcasted_iota(jnp.int32, (NB, G), 1)
        oh = (b_ref[...] == seg).astype(jnp.float32)
        part = lax.dot_general(oh, hn, (((0,), (0,)), ((), ())),
                               preferred_element_type=jnp.float32)
        cpart = lax.dot_general(oh, jnp.ones((NB, H), jnp.float32),
                                (((0,), (0,)), ((), ())),
                                preferred_element_type=jnp.float32)

        @pl.when(pid == 0)
        def _():
            rep_ref[...] = part
            cnt_ref[...] = cpart

        @pl.when(pid != 0)
        def _():
            rep_ref[...] += part
            cnt_ref[...] += cpart

        @pl.when(pid == grid[0] - 1)
        def _():
            rep_ref[...] = rep_ref[...] / jnp.maximum(cnt_ref[...], 1.0)

    full = lambda i: (0, 0)
    return pl.pallas_call(
        body,
        grid=grid,
        in_specs=[
            pl.BlockSpec((NB, H), lambda i: (i, 0)),
            pl.BlockSpec((2, NB, H), lambda i: (0, i, 0)),
            pl.BlockSpec((NB, 1), lambda i: (i, 0)),
            pl.BlockSpec((1, H), full),
            pl.BlockSpec((1, H), full),
            pl.BlockSpec((H, H), full),
        ],
        out_specs=[
            pl.BlockSpec((NB, H), lambda i: (i, 0)),
            pl.BlockSpec((NB, H), lambda i: (i, 0)),
            pl.BlockSpec((G, H), full),
        ],
        out_shape=[
            jax.ShapeDtypeStruct((N, H), jnp.float32),
            jax.ShapeDtypeStruct((N, H), jnp.float32),
            jax.ShapeDtypeStruct((G, H), jnp.float32),
        ],
        scratch_shapes=[pltpu.VMEM((G, H), jnp.float32)],
    )(h, a2, batch2, ng.reshape(1, H), nb.reshape(1, H), Wh_next)


def _tc_head(reps, hW1, hb1, hW2, hb2, hW3, hb3):
    def body(r0, r1, r2, r3, w1, b1, w2, b2, w3, b3, out_ref):
        rs = (r0, r1, r2, r3)
        z = b1[...]
        for l in range(L):
            z = z + jnp.dot(rs[l][...], w1[l], preferred_element_type=jnp.float32)
        z = _gelu(z)
        z = _gelu(jnp.dot(z, w2[...], preferred_element_type=jnp.float32) + b2[...])
        z = jnp.dot(z, w3[...], preferred_element_type=jnp.float32) + b3[...]
        out_ref[...] = z

    return pl.pallas_call(
        body,
        out_shape=jax.ShapeDtypeStruct((G, 1), jnp.float32),
    )(reps[0], reps[1], reps[2], reps[3], hW1.reshape(L, H, H), hb1.reshape(1, H),
      hW2, hb2.reshape(1, H // 2), hW3, hb3.reshape(1, 1))


def kernel(x, ei, ea, batch, emb_W, emb_b, emb_g, emb_bb, cW, cb, ng, nb,
           hW1, hb1, hW2, hb2, hW3, hb3):
    src = ei[0].astype(jnp.int32)
    dst = ei[1].astype(jnp.int32)
    batch2 = batch.astype(jnp.int32).reshape(N, 1)
    Wh = cW[:, :H, :]
    We = cW[:, H:, :]
    zstage = jnp.zeros((RSTG, H), jnp.float32)

    h, hw = _tc_embed(x, emb_W, emb_b, emb_g, emb_bb, Wh[0])
    reps = []
    for l in range(L):
        ewb = _tc_ewb(ea, We[l], cb[l])
        a2 = _sc_fused(hw, ewb, src, dst, zstage)
        h, hw, rep = _tc_update(h, a2, batch2, ng[l], nb[l], Wh[(l + 1) % L])
        reps.append(rep)
    z = _tc_head(reps, hW1, hb1, hW2, hb2, hW3, hb3)
    return z[:, 0]


# speedup vs baseline: 4.2290x; 4.2290x over previous
import functools

import jax
import jax.numpy as jnp
from jax import lax
from jax.experimental import pallas as pl
from jax.experimental.pallas import tpu as pltpu
from jax.experimental.pallas import tpu_sc as plsc

N = 10000
E = 320000
ND = 128
ED = 16
H = 128
L = 4
G = 64

NW = 32
EPW = E // NW
CK = 104
NCH = EPW // CK
CKT = EPW - NCH * CK
RSTG = 128
NRC = N // RSTG

_SQRT1_2 = 0.7071067811865476


def _gelu(x):
    return 0.5 * x * (1.0 + lax.erf(x * _SQRT1_2))


def _gelu_fast(x):
    z = jnp.abs(x) * _SQRT1_2
    t = 1.0 / (1.0 + 0.3275911 * z)
    poly = t * (0.254829592 + t * (-0.284496736 + t * (1.421413741
               + t * (-1.453152027 + t * 1.061405429))))
    erfabs = 1.0 - poly * jnp.exp(-z * z)
    return 0.5 * x * (1.0 + jnp.sign(x) * erfabs)


def _ln(x, g, b):
    mu = jnp.mean(x, axis=-1, keepdims=True)
    v = jnp.mean((x - mu) ** 2, axis=-1, keepdims=True)
    return (x - mu) * lax.rsqrt(v + 1e-5) * g + b


def _sc_gather(hw, src):
    mesh = plsc.VectorSubcoreMesh(core_axis_name="c", subcore_axis_name="s")

    @functools.partial(
        pl.kernel,
        out_type=jax.ShapeDtypeStruct((E, H), jnp.float32),
        mesh=mesh,
        scratch_types=[
            pltpu.VMEM((EPW,), jnp.int32),
            pltpu.VMEM((4, CK, H), jnp.float32),
            pltpu.VMEM((CKT, H), jnp.float32),
        ] + [pltpu.SemaphoreType.DMA] * 8,
    )
    def k(hw_hbm, src_hbm, out_hbm, idx_v, rows_v, tail_v, *sems):
        sem_g = sems[:4]
        sem_w = sems[4:]
        wid = lax.axis_index("s") * 2 + lax.axis_index("c")
        base = wid * EPW
        pltpu.sync_copy(src_hbm.at[pl.ds(base, EPW)], idx_v)

        def fire_gather(j, b):
            pltpu.async_copy(
                hw_hbm.at[idx_v.at[pl.ds(j * CK, CK)]], rows_v.at[b], sem_g[b])

        def drain_gather(j, b):
            pltpu.make_async_copy(
                hw_hbm.at[idx_v.at[pl.ds(j * CK, CK)]], rows_v.at[b],
                sem_g[b]).wait()

        def fire_write(j, b):
            pltpu.async_copy(
                rows_v.at[b], out_hbm.at[pl.ds(base + j * CK, CK)], sem_w[b])

        def drain_write(j, b):
            pltpu.make_async_copy(
                rows_v.at[b], out_hbm.at[pl.ds(base + j * CK, CK)],
                sem_w[b]).wait()

        fire_gather(0, 0)
        fire_gather(1, 1)

        def body(i, carry):
            for t in range(4):
                j = 4 * i + t
                b = t
                drain_gather(j, b)

                @pl.when(j >= 2)
                def _():
                    drain_write(j - 2, (b + 2) % 4)

                @pl.when(j + 2 < NCH)
                def _():
                    fire_gather(j + 2, (b + 2) % 4)

                fire_write(j, b)
            return carry

        lax.fori_loop(0, NCH // 4, body, 0)
        drain_write(NCH - 2, (NCH - 2) % 4)
        drain_write(NCH - 1, (NCH - 1) % 4)

        toff = NCH * CK
        pltpu.async_copy(
            hw_hbm.at[idx_v.at[pl.ds(toff, CKT)]], tail_v, sem_g[0]).wait()
        pltpu.sync_copy(tail_v, out_hbm.at[pl.ds(base + toff, CKT)])

    return k(hw, src)


def _sc_scatter(m, dst, zstage):
    mesh = plsc.VectorSubcoreMesh(core_axis_name="c", subcore_axis_name="s")

    @functools.partial(
        pl.kernel,
        out_type=jax.ShapeDtypeStruct((2, N, H), jnp.float32),
        mesh=mesh,
        scratch_types=[
            pltpu.VMEM((EPW,), jnp.int32),
            pltpu.VMEM((3, CK, H), jnp.float32),
            pltpu.VMEM_SHARED((N, H), jnp.float32),
        ] + [pltpu.SemaphoreType.DMA] * 6,
    )
    def k(m_hbm, dst_hbm, z_hbm, out_hbm, idx_v, rows_v, acc, *sems):
        sem_l = sems[:3]
        sem_s = sems[3:]
        c = lax.axis_index("c")
        s = lax.axis_index("s")
        wid = s * 2 + c
        base = wid * EPW

        for t in range(5):
            rc = t * 16 + s

            @pl.when(rc < NRC)
            def _():
                pltpu.sync_copy(z_hbm, acc.at[pl.ds(rc * RSTG, RSTG)])

        @pl.when(s == 0)
        def _():
            pltpu.sync_copy(z_hbm.at[pl.ds(0, N - NRC * RSTG)],
                            acc.at[pl.ds(NRC * RSTG, N - NRC * RSTG)])

        plsc.subcore_barrier()
        pltpu.sync_copy(dst_hbm.at[pl.ds(base, EPW)], idx_v)

        def fire_load(j, b):
            pltpu.async_copy(
                m_hbm.at[pl.ds(base + j * CK, CK)], rows_v.at[b], sem_l[b])

        def drain_load(j, b):
            pltpu.make_async_copy(
                m_hbm.at[pl.ds(base + j * CK, CK)], rows_v.at[b],
                sem_l[b]).wait()

        def fire_add(j, b):
            pltpu.async_copy(
                rows_v.at[b],
                acc.at[idx_v.at[pl.ds(j * CK, CK)]], sem_s[b], add=True)

        def drain_add(j, b):
            pltpu.make_async_copy(
                rows_v.at[b],
                acc.at[idx_v.at[pl.ds(j * CK, CK)]], sem_s[b]).wait()

        fire_load(0, 0)
        fire_load(1, 1)

        def body(i, carry):
            for t in range(3):
                j = 3 * i + t
                b = t
                drain_load(j, b)

                @pl.when(j >= 1)
                def _():
                    drain_add(j - 1, (b + 2) % 3)

                @pl.when(j + 2 < NCH)
                def _():
                    fire_load(j + 2, (b + 2) % 3)

                fire_add(j, b)
            return carry

        lax.fori_loop(0, NCH // 3, body, 0)
        drain_add(NCH - 1, (NCH - 1) % 3)

        toff = NCH * CK
        pltpu.sync_copy(m_hbm.at[pl.ds(base + toff, CKT)],
                        rows_v.at[0, pl.ds(0, CKT)])
        pltpu.async_copy(
            rows_v.at[0, pl.ds(0, CKT)],
            acc.at[idx_v.at[pl.ds(toff, CKT)]], sem_s[0], add=True).wait()
        plsc.subcore_barrier()

        for t in range(5):
            rc = t * 16 + s

            @pl.when(rc < NRC)
            def _():
                r0 = rc * RSTG
                pltpu.sync_copy(acc.at[pl.ds(r0, RSTG)],
                                out_hbm.at[c, pl.ds(r0, RSTG)])

        @pl.when(s == 1)
        def _():
            pltpu.sync_copy(acc.at[pl.ds(NRC * RSTG, N - NRC * RSTG)],
                            out_hbm.at[c, pl.ds(NRC * RSTG, N - NRC * RSTG)])

    return k(m, dst, zstage)


def _tc_embed(x, emb_W, emb_b, emb_g, emb_bb, Wh0):
    NB = 1000
    grid = (N // NB,)

    def body(x_ref, w_ref, b_ref, g_ref, bb_ref, wh_ref, h_ref, hw_ref):
        h = jnp.dot(x_ref[...], w_ref[...], preferred_element_type=jnp.float32)
        h = _gelu(_ln(h + b_ref[...], g_ref[...], bb_ref[...]))
        h_ref[...] = h
        hw_ref[...] = jnp.dot(h, wh_ref[...], preferred_element_type=jnp.float32)

    full = lambda i: (0, 0)
    return pl.pallas_call(
        body,
        grid=grid,
        in_specs=[
            pl.BlockSpec((NB, ND), lambda i: (i, 0)),
            pl.BlockSpec((ND, H), full),
            pl.BlockSpec((1, H), full),
            pl.BlockSpec((1, H), full),
            pl.BlockSpec((1, H), full),
            pl.BlockSpec((H, H), full),
        ],
        out_specs=[
            pl.BlockSpec((NB, H), lambda i: (i, 0)),
            pl.BlockSpec((NB, H), lambda i: (i, 0)),
        ],
        out_shape=[
            jax.ShapeDtypeStruct((N, H), jnp.float32),
            jax.ShapeDtypeStruct((N, H), jnp.float32),
        ],
    )(x, emb_W, emb_b.reshape(1, H), emb_g.reshape(1, H), emb_bb.reshape(1, H), Wh0)


def _tc_msg(hg, ea, We, cb):
    EB = 2560
    grid = (E // EB,)

    def body(hg_ref, ea_ref, we_ref, cb_ref, m_ref):
        s = hg_ref[...] + jnp.dot(ea_ref[...], we_ref[...],
                                  preferred_element_type=jnp.float32) + cb_ref[...]
        m_ref[...] = _gelu_fast(s)

    full = lambda i: (0, 0)
    return pl.pallas_call(
        body,
        grid=grid,
        in_specs=[
            pl.BlockSpec((EB, H), lambda i: (i, 0)),
            pl.BlockSpec((EB, ED), lambda i: (i, 0)),
            pl.BlockSpec((ED, H), full),
            pl.BlockSpec((1, H), full),
        ],
        out_specs=pl.BlockSpec((EB, H), lambda i: (i, 0)),
        out_shape=jax.ShapeDtypeStruct((E, H), jnp.float32),
    )(hg, ea, We, cb.reshape(1, H))


def _tc_update(h, a2, batch2, ng, nb, Wh_next):
    NB = 1000
    grid = (N // NB,)

    def body(h_ref, a_ref, b_ref, g_ref, bb_ref, wh_ref, hn_ref, hw_ref,
             rep_ref, cnt_ref):
        pid = pl.program_id(0)
        a = a_ref[0] + a_ref[1]
        hn = _ln(h_ref[...] + a, g_ref[...], bb_ref[...])
        hn_ref[...] = hn
        hw_ref[...] = jnp.dot(hn, wh_ref[...], preferred_element_type=jnp.float32)
        seg = lax.broadcasted_iota(jnp.int32, (NB, G), 1)
        oh = (b_ref[...] == seg).astype(jnp.float32)
        part = lax.dot_general(oh, hn, (((0,), (0,)), ((), ())),
                               preferred_element_type=jnp.float32)
        cpart = lax.dot_general(oh, jnp.ones((NB, H), jnp.float32),
                                (((0,), (0,)), ((), ())),
                                preferred_element_type=jnp.float32)

        @pl.when(pid == 0)
        def _():
            rep_ref[...] = part
            cnt_ref[...] = cpart

        @pl.when(pid != 0)
        def _():
            rep_ref[...] += part
            cnt_ref[...] += cpart

        @pl.when(pid == grid[0] - 1)
        def _():
            rep_ref[...] = rep_ref[...] / jnp.maximum(cnt_ref[...], 1.0)

    full = lambda i: (0, 0)
    return pl.pallas_call(
        body,
        grid=grid,
        in_specs=[
            pl.BlockSpec((NB, H), lambda i: (i, 0)),
            pl.BlockSpec((2, NB, H), lambda i: (0, i, 0)),
            pl.BlockSpec((NB, 1), lambda i: (i, 0)),
            pl.BlockSpec((1, H), full),
            pl.BlockSpec((1, H), full),
            pl.BlockSpec((H, H), full),
        ],
        out_specs=[
            pl.BlockSpec((NB, H), lambda i: (i, 0)),
            pl.BlockSpec((NB, H), lambda i: (i, 0)),
            pl.BlockSpec((G, H), full),
        ],
        out_shape=[
            jax.ShapeDtypeStruct((N, H), jnp.float32),
            jax.ShapeDtypeStruct((N, H), jnp.float32),
            jax.ShapeDtypeStruct((G, H), jnp.float32),
        ],
        scratch_shapes=[pltpu.VMEM((G, H), jnp.float32)],
    )(h, a2, batch2, ng.reshape(1, H), nb.reshape(1, H), Wh_next)


def _tc_head(reps, hW1, hb1, hW2, hb2, hW3, hb3):
    def body(r0, r1, r2, r3, w1, b1, w2, b2, w3, b3, out_ref):
        rs = (r0, r1, r2, r3)
        z = b1[...]
        for l in range(L):
            z = z + jnp.dot(rs[l][...], w1[l], preferred_element_type=jnp.float32)
        z = _gelu(z)
        z = _gelu(jnp.dot(z, w2[...], preferred_element_type=jnp.float32) + b2[...])
        z = jnp.dot(z, w3[...], preferred_element_type=jnp.float32) + b3[...]
        out_ref[...] = z

    return pl.pallas_call(
        body,
        out_shape=jax.ShapeDtypeStruct((G, 1), jnp.float32),
    )(reps[0], reps[1], reps[2], reps[3], hW1.reshape(L, H, H), hb1.reshape(1, H),
      hW2, hb2.reshape(1, H // 2), hW3, hb3.reshape(1, 1))


def kernel(x, ei, ea, batch, emb_W, emb_b, emb_g, emb_bb, cW, cb, ng, nb,
           hW1, hb1, hW2, hb2, hW3, hb3):
    src = ei[0].astype(jnp.int32)
    dst = ei[1].astype(jnp.int32)
    batch2 = batch.astype(jnp.int32).reshape(N, 1)
    Wh = cW[:, :H, :]
    We = cW[:, H:, :]
    zstage = jnp.zeros((RSTG, H), jnp.float32)

    h, hw = _tc_embed(x, emb_W, emb_b, emb_g, emb_bb, Wh[0])
    reps = []
    for l in range(L):
        hg = _sc_gather(hw, src)
        m = _tc_msg(hg, ea, We[l], cb[l])
        a2 = _sc_scatter(m, dst, zstage)
        h, hw, rep = _tc_update(h, a2, batch2, ng[l], nb[l], Wh[(l + 1) % L])
        reps.append(rep)
    z = _tc_head(reps, hW1, hb1, hW2, hb2, hW3, hb3)
    return z[:, 0]


# speedup vs baseline: 4.7116x; 1.1141x over previous
import functools

import jax
import jax.numpy as jnp
from jax import lax
from jax.experimental import pallas as pl
from jax.experimental.pallas import tpu as pltpu
from jax.experimental.pallas import tpu_sc as plsc

N = 10000
E = 320000
ND = 128
ED = 16
H = 128
L = 4
G = 64

NW = 32
E2 = E // 2
EPW = E2 // NW
CK = 104
NCH = EPW // CK
CKT = EPW - NCH * CK
RSTG = 128
NRC = N // RSTG

_SQRT1_2 = 0.7071067811865476


def _gelu(x):
    return 0.5 * x * (1.0 + lax.erf(x * _SQRT1_2))


def _ln(x, g, b):
    mu = jnp.mean(x, axis=-1, keepdims=True)
    v = jnp.mean((x - mu) ** 2, axis=-1, keepdims=True)
    return (x - mu) * lax.rsqrt(v + 1e-5) * g + b


def _sc_gather(hw, src, half):
    mesh = plsc.VectorSubcoreMesh(core_axis_name="c", subcore_axis_name="s")

    @functools.partial(
        pl.kernel,
        out_type=jax.ShapeDtypeStruct((E2, H), jnp.float32),
        mesh=mesh,
        scratch_types=[
            pltpu.VMEM((EPW,), jnp.int32),
            pltpu.VMEM((4, CK, H), jnp.float32),
            pltpu.VMEM((CKT, H), jnp.float32),
        ] + [pltpu.SemaphoreType.DMA] * 8,
    )
    def k(hw_hbm, src_hbm, out_hbm, idx_v, rows_v, tail_v, *sems):
        sem_g = sems[:4]
        sem_w = sems[4:]
        wid = lax.axis_index("s") * 2 + lax.axis_index("c")
        base = wid * EPW
        pltpu.sync_copy(src_hbm.at[pl.ds(half * E2 + base, EPW)], idx_v)

        def fire_gather(j, b):
            pltpu.async_copy(
                hw_hbm.at[idx_v.at[pl.ds(j * CK, CK)]], rows_v.at[b], sem_g[b])

        def drain_gather(j, b):
            pltpu.make_async_copy(
                hw_hbm.at[idx_v.at[pl.ds(j * CK, CK)]], rows_v.at[b],
                sem_g[b]).wait()

        def fire_write(j, b):
            pltpu.async_copy(
                rows_v.at[b], out_hbm.at[pl.ds(base + j * CK, CK)], sem_w[b])

        def drain_write(j, b):
            pltpu.make_async_copy(
                rows_v.at[b], out_hbm.at[pl.ds(base + j * CK, CK)],
                sem_w[b]).wait()

        fire_gather(0, 0)
        fire_gather(1, 1)

        def body(i, carry):
            for t in range(4):
                j = 4 * i + t
                b = t
                drain_gather(j, b)

                @pl.when(j >= 2)
                def _():
                    drain_write(j - 2, (b + 2) % 4)

                @pl.when(j + 2 < NCH)
                def _():
                    fire_gather(j + 2, (b + 2) % 4)

                fire_write(j, b)
            return carry

        lax.fori_loop(0, NCH // 4, body, 0)
        drain_write(NCH - 2, (NCH - 2) % 4)
        drain_write(NCH - 1, (NCH - 1) % 4)

        toff = NCH * CK
        pltpu.async_copy(
            hw_hbm.at[idx_v.at[pl.ds(toff, CKT)]], tail_v, sem_g[0]).wait()
        pltpu.sync_copy(tail_v, out_hbm.at[pl.ds(base + toff, CKT)])

    return k(hw, src)


def _sc_scatter(m, dst, half, zstage):
    mesh = plsc.VectorSubcoreMesh(core_axis_name="c", subcore_axis_name="s")

    @functools.partial(
        pl.kernel,
        out_type=jax.ShapeDtypeStruct((2, N, H), jnp.float32),
        mesh=mesh,
        scratch_types=[
            pltpu.VMEM((EPW,), jnp.int32),
            pltpu.VMEM((3, CK, H), jnp.float32),
            pltpu.VMEM_SHARED((N, H), jnp.float32),
        ] + [pltpu.SemaphoreType.DMA] * 6,
    )
    def k(m_hbm, dst_hbm, z_hbm, out_hbm, idx_v, rows_v, acc, *sems):
        sem_l = sems[:3]
        sem_s = sems[3:]
        c = lax.axis_index("c")
        s = lax.axis_index("s")
        wid = s * 2 + c
        base = wid * EPW

        for t in range(5):
            rc = t * 16 + s

            @pl.when(rc < NRC)
            def _():
                pltpu.sync_copy(z_hbm, acc.at[pl.ds(rc * RSTG, RSTG)])

        @pl.when(s == 0)
        def _():
            pltpu.sync_copy(z_hbm.at[pl.ds(0, N - NRC * RSTG)],
                            acc.at[pl.ds(NRC * RSTG, N - NRC * RSTG)])

        plsc.subcore_barrier()
        pltpu.sync_copy(dst_hbm.at[pl.ds(half * E2 + base, EPW)], idx_v)

        def fire_load(j, b):
            pltpu.async_copy(
                m_hbm.at[pl.ds(base + j * CK, CK)], rows_v.at[b], sem_l[b])

        def drain_load(j, b):
            pltpu.make_async_copy(
                m_hbm.at[pl.ds(base + j * CK, CK)], rows_v.at[b],
                sem_l[b]).wait()

        def fire_add(j, b):
            pltpu.async_copy(
                rows_v.at[b],
                acc.at[idx_v.at[pl.ds(j * CK, CK)]], sem_s[b], add=True)

        def drain_add(j, b):
            pltpu.make_async_copy(
                rows_v.at[b],
                acc.at[idx_v.at[pl.ds(j * CK, CK)]], sem_s[b]).wait()

        fire_load(0, 0)
        fire_load(1, 1)

        def body(i, carry):
            for t in range(3):
                j = 3 * i + t
                b = t
                drain_load(j, b)

                @pl.when(j >= 1)
                def _():
                    drain_add(j - 1, (b + 2) % 3)

                @pl.when(j + 2 < NCH)
                def _():
                    fire_load(j + 2, (b + 2) % 3)

                fire_add(j, b)
            return carry

        lax.fori_loop(0, NCH // 3, body, 0)
        drain_add(NCH - 1, (NCH - 1) % 3)

        toff = NCH * CK
        pltpu.sync_copy(m_hbm.at[pl.ds(base + toff, CKT)],
                        rows_v.at[0, pl.ds(0, CKT)])
        pltpu.async_copy(
            rows_v.at[0, pl.ds(0, CKT)],
            acc.at[idx_v.at[pl.ds(toff, CKT)]], sem_s[0], add=True).wait()
        plsc.subcore_barrier()

        for t in range(5):
            rc = t * 16 + s

            @pl.when(rc < NRC)
            def _():
                r0 = rc * RSTG
                pltpu.sync_copy(acc.at[pl.ds(r0, RSTG)],
                                out_hbm.at[c, pl.ds(r0, RSTG)])

        @pl.when(s == 1)
        def _():
            pltpu.sync_copy(acc.at[pl.ds(NRC * RSTG, N - NRC * RSTG)],
                            out_hbm.at[c, pl.ds(NRC * RSTG, N - NRC * RSTG)])

    return k(m, dst, zstage)


def _tc_embed(x, emb_W, emb_b, emb_g, emb_bb, Wh0):
    NB = 1000
    grid = (N // NB,)

    def body(x_ref, w_ref, b_ref, g_ref, bb_ref, wh_ref, h_ref, hw_ref):
        h = jnp.dot(x_ref[...], w_ref[...], preferred_element_type=jnp.float32)
        h = _gelu(_ln(h + b_ref[...], g_ref[...], bb_ref[...]))
        h_ref[...] = h
        hw_ref[...] = jnp.dot(h, wh_ref[...], preferred_element_type=jnp.float32)

    full = lambda i: (0, 0)
    return pl.pallas_call(
        body,
        grid=grid,
        in_specs=[
            pl.BlockSpec((NB, ND), lambda i: (i, 0)),
            pl.BlockSpec((ND, H), full),
            pl.BlockSpec((1, H), full),
            pl.BlockSpec((1, H), full),
            pl.BlockSpec((1, H), full),
            pl.BlockSpec((H, H), full),
        ],
        out_specs=[
            pl.BlockSpec((NB, H), lambda i: (i, 0)),
            pl.BlockSpec((NB, H), lambda i: (i, 0)),
        ],
        out_shape=[
            jax.ShapeDtypeStruct((N, H), jnp.float32),
            jax.ShapeDtypeStruct((N, H), jnp.float32),
        ],
    )(x, emb_W, emb_b.reshape(1, H), emb_g.reshape(1, H), emb_bb.reshape(1, H), Wh0)


def _tc_msg(hg, ea, We, cb, half):
    EB = 2000
    grid = (E2 // EB,)
    off = half * (E2 // EB)

    def body(hg_ref, ea_ref, we_ref, cb_ref, m_ref):
        s = hg_ref[...] + jnp.dot(ea_ref[...], we_ref[...],
                                  preferred_element_type=jnp.float32) + cb_ref[...]
        m_ref[...] = _gelu(s)

    full = lambda i: (0, 0)
    return pl.pallas_call(
        body,
        grid=grid,
        in_specs=[
            pl.BlockSpec((EB, H), lambda i: (i, 0)),
            pl.BlockSpec((EB, ED), lambda i: (i + off, 0)),
            pl.BlockSpec((ED, H), full),
            pl.BlockSpec((1, H), full),
        ],
        out_specs=pl.BlockSpec((EB, H), lambda i: (i, 0)),
        out_shape=jax.ShapeDtypeStruct((E2, H), jnp.float32),
    )(hg, ea, We, cb.reshape(1, H))


def _tc_update(h, a2a, a2b, batch2, ng, nb, Wh_next):
    NB = 1000
    grid = (N // NB,)

    def body(h_ref, a_ref, a2_ref, b_ref, g_ref, bb_ref, wh_ref, hn_ref, hw_ref,
             rep_ref, cnt_ref):
        pid = pl.program_id(0)
        a = (a_ref[0] + a_ref[1]) + (a2_ref[0] + a2_ref[1])
        hn = _ln(h_ref[...] + a, g_ref[...], bb_ref[...])
        hn_ref[...] = hn
        hw_ref[...] = jnp.dot(hn, wh_ref[...], preferred_element_type=jnp.float32)
        seg = lax.broadcasted_iota(jnp.int32, (NB, G), 1)
        oh = (b_ref[...] == seg).astype(jnp.float32)
        part = lax.dot_general(oh, hn, (((0,), (0,)), ((), ())),
                               preferred_element_type=jnp.float32)
        cpart = lax.dot_general(oh, jnp.ones((NB, H), jnp.float32),
                                (((0,), (0,)), ((), ())),
                                preferred_element_type=jnp.float32)

        @pl.when(pid == 0)
        def _():
            rep_ref[...] = part
            cnt_ref[...] = cpart

        @pl.when(pid != 0)
        def _():
            rep_ref[...] += part
            cnt_ref[...] += cpart

        @pl.when(pid == grid[0] - 1)
        def _():
            rep_ref[...] = rep_ref[...] / jnp.maximum(cnt_ref[...], 1.0)

    full = lambda i: (0, 0)
    return pl.pallas_call(
        body,
        grid=grid,
        in_specs=[
            pl.BlockSpec((NB, H), lambda i: (i, 0)),
            pl.BlockSpec((2, NB, H), lambda i: (0, i, 0)),
            pl.BlockSpec((2, NB, H), lambda i: (0, i, 0)),
            pl.BlockSpec((NB, 1), lambda i: (i, 0)),
            pl.BlockSpec((1, H), full),
            pl.BlockSpec((1, H), full),
            pl.BlockSpec((H, H), full),
        ],
        out_specs=[
            pl.BlockSpec((NB, H), lambda i: (i, 0)),
            pl.BlockSpec((NB, H), lambda i: (i, 0)),
            pl.BlockSpec((G, H), full),
        ],
        out_shape=[
            jax.ShapeDtypeStruct((N, H), jnp.float32),
            jax.ShapeDtypeStruct((N, H), jnp.float32),
            jax.ShapeDtypeStruct((G, H), jnp.float32),
        ],
        scratch_shapes=[pltpu.VMEM((G, H), jnp.float32)],
    )(h, a2a, a2b, batch2, ng.reshape(1, H), nb.reshape(1, H), Wh_next)


def _tc_head(reps, hW1, hb1, hW2, hb2, hW3, hb3):
    def body(r0, r1, r2, r3, w1, b1, w2, b2, w3, b3, out_ref):
        rs = (r0, r1, r2, r3)
        z = b1[...]
        for l in range(L):
            z = z + jnp.dot(rs[l][...], w1[l], preferred_element_type=jnp.float32)
        z = _gelu(z)
        z = _gelu(jnp.dot(z, w2[...], preferred_element_type=jnp.float32) + b2[...])
        z = jnp.dot(z, w3[...], preferred_element_type=jnp.float32) + b3[...]
        out_ref[...] = z

    return pl.pallas_call(
        body,
        out_shape=jax.ShapeDtypeStruct((G, 1), jnp.float32),
    )(reps[0], reps[1], reps[2], reps[3], hW1.reshape(L, H, H), hb1.reshape(1, H),
      hW2, hb2.reshape(1, H // 2), hW3, hb3.reshape(1, 1))


def kernel(x, ei, ea, batch, emb_W, emb_b, emb_g, emb_bb, cW, cb, ng, nb,
           hW1, hb1, hW2, hb2, hW3, hb3):
    src = ei[0].astype(jnp.int32)
    dst = ei[1].astype(jnp.int32)
    batch2 = batch.astype(jnp.int32).reshape(N, 1)
    Wh = cW[:, :H, :]
    We = cW[:, H:, :]
    zstage = jnp.zeros((RSTG, H), jnp.float32)

    h, hw = _tc_embed(x, emb_W, emb_b, emb_g, emb_bb, Wh[0])
    reps = []
    for l in range(L):
        hga = _sc_gather(hw, src, 0)
        ma = _tc_msg(hga, ea, We[l], cb[l], 0)
        hgb = _sc_gather(hw, src, 1)
        mb = _tc_msg(hgb, ea, We[l], cb[l], 1)
        a2a = _sc_scatter(ma, dst, 0, zstage)
        a2b = _sc_scatter(mb, dst, 1, zstage)
        h, hw, rep = _tc_update(h, a2a, a2b, batch2, ng[l], nb[l],
                                Wh[(l + 1) % L])
        reps.append(rep)
    z = _tc_head(reps, hW1, hb1, hW2, hb2, hW3, hb3)
    return z[:, 0]
